# baseline (speedup 1.0000x reference)
import functools

import jax
import jax.numpy as jnp
from jax import lax
from jax.experimental import pallas as pl
from jax.experimental.pallas import tpu as pltpu
from jax.experimental.pallas import tpu_sc as plsc

NUM_FIELD = 26
EMB = 16
LIN_DIM = 10
FE_W = 512
TR_CH = 7936
MAIN_ROWS = 999936
PACK = 128 // EMB

_NC = 2
_NS = 16
_NW = _NC * _NS
_CHUNK = 128


def _tr_body(in_ref, out_ref):
    x = in_ref[...]
    y = jnp.transpose(x.reshape(EMB, TR_CH // PACK, PACK), (1, 2, 0))
    out_ref[...] = y.reshape(TR_CH // PACK, 128)


def _linearize(table_t):
    grid = MAIN_ROWS // TR_CH
    return pl.pallas_call(
        _tr_body,
        grid=(grid,),
        in_specs=[pl.BlockSpec((EMB, TR_CH), lambda i: (0, i))],
        out_specs=pl.BlockSpec((TR_CH // PACK, 128), lambda i: (i, 0)),
        out_shape=jax.ShapeDtypeStruct((MAIN_ROWS // PACK, 128), jnp.float32),
    )(table_t)


def _sc_gather(idx3, main_lin, tail_lin, batch):
    rows_per_w = batch // _NW
    n_chunks = rows_per_w * NUM_FIELD // _CHUNK
    main_packed = MAIN_ROWS // PACK
    buf_rows = _CHUNK + PACK
    mesh = plsc.VectorSubcoreMesh(
        core_axis_name="c", subcore_axis_name="s",
        num_cores=_NC, num_subcores=_NS)

    @functools.partial(
        pl.kernel,
        out_type=jax.ShapeDtypeStruct((batch, FE_W), jnp.float32),
        mesh=mesh,
        compiler_params=pltpu.CompilerParams(needs_layout_passes=False),
        scratch_types=[
            pltpu.VMEM((n_chunks, _CHUNK), jnp.int32),
            pltpu.VMEM((2, _CHUNK), jnp.int32),
            pltpu.VMEM((2, buf_rows, 128), jnp.float32),
            pltpu.VMEM((rows_per_w, FE_W), jnp.float32),
            pltpu.SemaphoreType.DMA,
            pltpu.SemaphoreType.DMA,
            pltpu.SemaphoreType.DMA,
        ],
    )
    def gather_kernel(idx_hbm, main_hbm, tail_hbm, out_hbm,
                      idx_v, idxr_v, buf_v, rows_v, s0, s1, s2):
        wid = lax.axis_index("s") * _NC + lax.axis_index("c")
        base = wid * rows_per_w
        pltpu.sync_copy(idx_hbm.at[wid], idx_v)
        pltpu.sync_copy(tail_hbm, buf_v.at[0, pl.ds(_CHUNK, PACK)])
        pltpu.sync_copy(tail_hbm, buf_v.at[1, pl.ds(_CHUNK, PACK)])
        lanes = lax.iota(jnp.int32, 16)
        sems = [s0, s1]

        def prep(j, jb):
            for g in range(_CHUNK // 16):
                r = idx_v[j, pl.ds(16 * g, 16)]
                rp = jnp.minimum(r >> 3, main_packed - 1)
                idxr_v[jb, pl.ds(16 * g, 16)] = rp
            pltpu.async_copy(
                main_hbm.at[idxr_v.at[jb]],
                buf_v.at[jb, pl.ds(0, _CHUNK)],
                sems[jb])

        def wait_buf(jb):
            pltpu.make_async_copy(
                main_hbm.at[pl.ds(0, _CHUNK)],
                buf_v.at[jb, pl.ds(0, _CHUNK)],
                sems[jb]).wait()

        def extract(j, jb):
            for g in range(_CHUNK // 16):
                l0 = 16 * g
                r = idx_v[j, pl.ds(l0, 16)]
                rp = r >> 3
                sub = (r & 7) << 4
                row = jnp.where(rp >= main_packed,
                                _CHUNK + rp - main_packed,
                                lanes + l0)
                p = j * _CHUNK + l0 + lanes
                bbv = p // NUM_FIELD
                cov = (p % NUM_FIELD) * EMB
                jbv = jnp.full((16,), jb, jnp.int32)
                for m in range(EMB):
                    vals = plsc.load_gather(buf_v, [jbv, row, sub + m])
                    plsc.store_scatter(rows_v, [bbv, cov + m], vals)

        prep(0, 0)

        def pair_body(k, _):
            j0 = 2 * k
            prep(j0 + 1, 1)
            wait_buf(0)
            extract(j0, 0)

            @pl.when(j0 + 2 < n_chunks)
            def _():
                prep(j0 + 2, 0)

            wait_buf(1)
            extract(j0 + 1, 1)
            return 0

        lax.fori_loop(0, n_chunks // 2, pair_body, 0)
        pltpu.async_copy(rows_v, out_hbm.at[pl.ds(base, rows_per_w)],
                         s2).wait()

    return gather_kernel(idx3, main_lin, tail_lin)


def _tc_body(fe_ref, wl_ref, theta_ref, w1a_ref, w1b_ref, b1_ref, g1_ref,
             be1_ref, w2_ref, b2_ref, g2_ref, be2_ref, wfc_ref, bfc_ref,
             out_ref):
    f32 = jnp.float32
    fe = fe_ref[...]
    lz = jnp.dot(fe, wl_ref[...], preferred_element_type=f32)
    fe2 = fe * fe
    row = lax.broadcasted_iota(jnp.int32, (FE_W, NUM_FIELD), 0)
    col = lax.broadcasted_iota(jnp.int32, (FE_W, NUM_FIELD), 1)
    sel = jnp.where(row // EMB == col, 1.0, 0.0).astype(f32)
    s = jnp.dot(fe2, sel, preferred_element_type=f32)
    th = theta_ref[...]
    th2 = th * th
    lp = lax.dot_general(s, th2, (((1,), (1,)), ((), ())),
                         preferred_element_type=f32)

    def bn_relu(y, g, b):
        mean = jnp.mean(y, axis=0, keepdims=True)
        var = jnp.mean((y - mean) ** 2, axis=0, keepdims=True)
        return jnp.maximum(g * (y - mean) / jnp.sqrt(var + 1e-5) + b, 0.0)

    h = (jnp.dot(lz, w1a_ref[...], preferred_element_type=f32)
         + jnp.dot(lp, w1b_ref[...], preferred_element_type=f32)
         + b1_ref[...])
    h = bn_relu(h, g1_ref[...], be1_ref[...])
    h = jnp.dot(h, w2_ref[...], preferred_element_type=f32) + b2_ref[...]
    h = bn_relu(h, g2_ref[...], be2_ref[...])
    out_ref[...] = (jnp.dot(h, wfc_ref[...], preferred_element_type=f32)
                    + bfc_ref[...])


def _tc_compute(fe, wl, theta, w1a, w1b, b1, g1, be1, w2, b2, g2, be2,
                wfc, bfc, interpret=False):
    batch = fe.shape[0]
    return pl.pallas_call(
        _tc_body,
        out_shape=jax.ShapeDtypeStruct((batch, 1), jnp.float32),
        interpret=interpret,
    )(fe, wl, theta, w1a, w1b, b1, g1, be1, w2, b2, g2, be2, wfc, bfc)


def kernel(feat_index, feat_value, emb_table, linear_weights, theta,
           W1, b1, g1, be1, W2, b2, g2, be2, Wfc, bfc):
    del feat_value
    batch = feat_index.shape[0]
    rows_per_w = batch // _NW

    table_t = emb_table.T
    main_lin = _linearize(table_t)
    tail_lin = table_t[:, MAIN_ROWS:].T.reshape(PACK, 128)

    idx3 = feat_index.astype(jnp.int32).reshape(
        _NW, NUM_FIELD * rows_per_w // _CHUNK, _CHUNK)
    fe = _sc_gather(idx3, main_lin, tail_lin, batch)

    wl = linear_weights.reshape(LIN_DIM, NUM_FIELD * EMB).T
    wl = jnp.pad(wl, ((0, FE_W - NUM_FIELD * EMB), (0, 0)))
    return _tc_compute(
        fe, wl, theta,
        W1[:LIN_DIM], W1[LIN_DIM:],
        b1.reshape(1, -1), g1.reshape(1, -1), be1.reshape(1, -1),
        W2, b2.reshape(1, -1), g2.reshape(1, -1), be2.reshape(1, -1),
        Wfc, bfc.reshape(1, 1))


# speedup vs baseline: 1.9823x; 1.9823x over previous
import functools

import jax
import jax.numpy as jnp
from jax import lax
from jax.experimental import pallas as pl
from jax.experimental.pallas import tpu as pltpu
from jax.experimental.pallas import tpu_sc as plsc

NUM_FIELD = 26
EMB = 16
LIN_DIM = 10
FE_W = 512
TR_CH = 7936
MAIN_ROWS = 999936
PACK = 128 // EMB

_NC = 2
_NS = 16
_NW = _NC * _NS
_CHUNK = 128


_BLK = 7
_NBLK = MAIN_ROWS // (128 * _BLK)
_BLK_W = 128 * _BLK
_BLK_R = _BLK_W // PACK


def _linearize(table_t):
    mesh = plsc.VectorSubcoreMesh(
        core_axis_name="c", subcore_axis_name="s",
        num_cores=_NC, num_subcores=_NS)
    blocks_per_w = (_NBLK + _NW - 1) // _NW

    @functools.partial(
        pl.kernel,
        out_type=jax.ShapeDtypeStruct((MAIN_ROWS // PACK, 128), jnp.float32),
        mesh=mesh,
        compiler_params=pltpu.CompilerParams(needs_layout_passes=False),
        scratch_types=[
            pltpu.VMEM((2, EMB, _BLK_W), jnp.float32),
            pltpu.VMEM((2, _BLK_R, 128), jnp.float32),
            pltpu.SemaphoreType.DMA,
            pltpu.SemaphoreType.DMA,
            pltpu.SemaphoreType.DMA,
            pltpu.SemaphoreType.DMA,
        ],
    )
    def lin_kernel(tt_hbm, out_hbm, in_v, out_v, si0, si1, so0, so1):
        wid = lax.axis_index("s") * _NC + lax.axis_index("c")
        lanes = lax.iota(jnp.int32, 16)
        sin = [si0, si1]
        sout = [so0, so1]

        def blk_id(b):
            return wid + _NW * b

        def fetch(b, pb):
            @pl.when(blk_id(b) < _NBLK)
            def _():
                pltpu.async_copy(
                    tt_hbm.at[:, pl.ds(blk_id(b) * _BLK_W, _BLK_W)],
                    in_v.at[pb], sin[pb])

        def wait_in(pb):
            pltpu.make_async_copy(
                tt_hbm.at[:, pl.ds(0, _BLK_W)], in_v.at[pb],
                sin[pb]).wait()

        def assemble(b, pb):
            @pl.when(blk_id(b) < _NBLK)
            def _():
                def row_body(rl, _):
                    for j in range(PACK):
                        vals = plsc.load_gather(
                            in_v,
                            [jnp.full((16,), pb, jnp.int32), lanes,
                             jnp.full((16,), rl * PACK + j, jnp.int32)])
                        out_v[pb, rl, pl.ds(j * EMB, EMB)] = vals
                    return 0

                lax.fori_loop(0, _BLK_R, row_body, 0)
                pltpu.async_copy(
                    out_v.at[pb],
                    out_hbm.at[pl.ds(blk_id(b) * _BLK_R, _BLK_R)],
                    sout[pb])

        def wait_out(pb):
            pltpu.make_async_copy(
                out_hbm.at[pl.ds(0, _BLK_R)], out_v.at[pb],
                sout[pb]).wait()

        fetch(0, 0)

        def loop_body(k, _):
            b0 = 2 * k
            fetch(b0 + 1, 1)

            @pl.when(blk_id(b0) < _NBLK)
            def _():
                wait_in(0)

            @pl.when((b0 >= 2) & (blk_id(b0 - 2) < _NBLK))
            def _():
                wait_out(0)

            assemble(b0, 0)
            fetch(b0 + 2, 0)

            @pl.when(blk_id(b0 + 1) < _NBLK)
            def _():
                wait_in(1)

            @pl.when((b0 >= 1) & (blk_id(b0 - 1) < _NBLK))
            def _():
                wait_out(1)

            assemble(b0 + 1, 1)
            return 0

        n_pairs = (blocks_per_w + 1) // 2
        lax.fori_loop(0, n_pairs, loop_body, 0)
        @pl.when(blk_id(2 * n_pairs - 2) < _NBLK)
        def _():
            wait_out(0)

        @pl.when(blk_id(2 * n_pairs - 1) < _NBLK)
        def _():
            wait_out(1)

    return lin_kernel(table_t)


def _sc_gather(idx3, main_lin, tail_lin, batch):
    rows_per_w = batch // _NW
    n_chunks = rows_per_w * NUM_FIELD // _CHUNK
    main_packed = MAIN_ROWS // PACK
    buf_rows = _CHUNK + PACK
    mesh = plsc.VectorSubcoreMesh(
        core_axis_name="c", subcore_axis_name="s",
        num_cores=_NC, num_subcores=_NS)

    @functools.partial(
        pl.kernel,
        out_type=jax.ShapeDtypeStruct((batch, FE_W), jnp.float32),
        mesh=mesh,
        compiler_params=pltpu.CompilerParams(needs_layout_passes=False),
        scratch_types=[
            pltpu.VMEM((n_chunks, _CHUNK), jnp.int32),
            pltpu.VMEM((2, _CHUNK), jnp.int32),
            pltpu.VMEM((2, buf_rows, 128), jnp.float32),
            pltpu.VMEM((rows_per_w, FE_W), jnp.float32),
            pltpu.SemaphoreType.DMA,
            pltpu.SemaphoreType.DMA,
            pltpu.SemaphoreType.DMA,
        ],
    )
    def gather_kernel(idx_hbm, main_hbm, tail_hbm, out_hbm,
                      idx_v, idxr_v, buf_v, rows_v, s0, s1, s2):
        wid = lax.axis_index("s") * _NC + lax.axis_index("c")
        base = wid * rows_per_w
        pltpu.sync_copy(idx_hbm.at[wid], idx_v)
        pltpu.sync_copy(tail_hbm, buf_v.at[0, pl.ds(_CHUNK, PACK)])
        pltpu.sync_copy(tail_hbm, buf_v.at[1, pl.ds(_CHUNK, PACK)])
        lanes = lax.iota(jnp.int32, 16)
        sems = [s0, s1]

        def prep(j, jb):
            for g in range(_CHUNK // 16):
                r = idx_v[j, pl.ds(16 * g, 16)]
                rp = jnp.minimum(r >> 3, main_packed - 1)
                idxr_v[jb, pl.ds(16 * g, 16)] = rp
            pltpu.async_copy(
                main_hbm.at[idxr_v.at[jb]],
                buf_v.at[jb, pl.ds(0, _CHUNK)],
                sems[jb])

        def wait_buf(jb):
            pltpu.make_async_copy(
                main_hbm.at[pl.ds(0, _CHUNK)],
                buf_v.at[jb, pl.ds(0, _CHUNK)],
                sems[jb]).wait()

        def extract(j, jb):
            for g in range(_CHUNK // 16):
                l0 = 16 * g
                r = idx_v[j, pl.ds(l0, 16)]
                rp = r >> 3
                sub = (r & 7) << 4
                row = jnp.where(rp >= main_packed,
                                _CHUNK + rp - main_packed,
                                lanes + l0)
                p = j * _CHUNK + l0 + lanes
                bbv = p // NUM_FIELD
                cov = (p % NUM_FIELD) * EMB
                jbv = jnp.full((16,), jb, jnp.int32)
                for m in range(EMB):
                    vals = plsc.load_gather(buf_v, [jbv, row, sub + m])
                    plsc.store_scatter(rows_v, [bbv, cov + m], vals)

        prep(0, 0)

        def pair_body(k, _):
            j0 = 2 * k
            prep(j0 + 1, 1)
            wait_buf(0)
            extract(j0, 0)

            @pl.when(j0 + 2 < n_chunks)
            def _():
                prep(j0 + 2, 0)

            wait_buf(1)
            extract(j0 + 1, 1)
            return 0

        lax.fori_loop(0, n_chunks // 2, pair_body, 0)
        pltpu.async_copy(rows_v, out_hbm.at[pl.ds(base, rows_per_w)],
                         s2).wait()

    return gather_kernel(idx3, main_lin, tail_lin)


def _tc_body(fe_ref, wl_ref, theta_ref, w1a_ref, w1b_ref, b1_ref, g1_ref,
             be1_ref, w2_ref, b2_ref, g2_ref, be2_ref, wfc_ref, bfc_ref,
             out_ref):
    f32 = jnp.float32
    fe = fe_ref[...]
    lz = jnp.dot(fe, wl_ref[...], preferred_element_type=f32)
    fe2 = fe * fe
    row = lax.broadcasted_iota(jnp.int32, (FE_W, NUM_FIELD), 0)
    col = lax.broadcasted_iota(jnp.int32, (FE_W, NUM_FIELD), 1)
    sel = jnp.where(row // EMB == col, 1.0, 0.0).astype(f32)
    s = jnp.dot(fe2, sel, preferred_element_type=f32)
    th = theta_ref[...]
    th2 = th * th
    lp = lax.dot_general(s, th2, (((1,), (1,)), ((), ())),
                         preferred_element_type=f32)

    def bn_relu(y, g, b):
        mean = jnp.mean(y, axis=0, keepdims=True)
        var = jnp.mean((y - mean) ** 2, axis=0, keepdims=True)
        return jnp.maximum(g * (y - mean) / jnp.sqrt(var + 1e-5) + b, 0.0)

    h = (jnp.dot(lz, w1a_ref[...], preferred_element_type=f32)
         + jnp.dot(lp, w1b_ref[...], preferred_element_type=f32)
         + b1_ref[...])
    h = bn_relu(h, g1_ref[...], be1_ref[...])
    h = jnp.dot(h, w2_ref[...], preferred_element_type=f32) + b2_ref[...]
    h = bn_relu(h, g2_ref[...], be2_ref[...])
    out_ref[...] = (jnp.dot(h, wfc_ref[...], preferred_element_type=f32)
                    + bfc_ref[...])


def _tc_compute(fe, wl, theta, w1a, w1b, b1, g1, be1, w2, b2, g2, be2,
                wfc, bfc, interpret=False):
    batch = fe.shape[0]
    return pl.pallas_call(
        _tc_body,
        out_shape=jax.ShapeDtypeStruct((batch, 1), jnp.float32),
        interpret=interpret,
    )(fe, wl, theta, w1a, w1b, b1, g1, be1, w2, b2, g2, be2, wfc, bfc)


def kernel(feat_index, feat_value, emb_table, linear_weights, theta,
           W1, b1, g1, be1, W2, b2, g2, be2, Wfc, bfc):
    del feat_value
    batch = feat_index.shape[0]
    rows_per_w = batch // _NW

    table_t = emb_table.T
    main_lin = _linearize(table_t)
    tail_lin = table_t[:, MAIN_ROWS:].T.reshape(PACK, 128)

    idx3 = feat_index.astype(jnp.int32).reshape(
        _NW, NUM_FIELD * rows_per_w // _CHUNK, _CHUNK)
    fe = _sc_gather(idx3, main_lin, tail_lin, batch)

    wl = linear_weights.reshape(LIN_DIM, NUM_FIELD * EMB).T
    wl = jnp.pad(wl, ((0, FE_W - NUM_FIELD * EMB), (0, 0)))
    return _tc_compute(
        fe, wl, theta,
        W1[:LIN_DIM], W1[LIN_DIM:],
        b1.reshape(1, -1), g1.reshape(1, -1), be1.reshape(1, -1),
        W2, b2.reshape(1, -1), g2.reshape(1, -1), be2.reshape(1, -1),
        Wfc, bfc.reshape(1, 1))
